# baseline (speedup 1.0000x reference)
import functools

import jax
import jax.numpy as jnp
from jax import lax
from jax.experimental import pallas as pl
from jax.experimental.pallas import tpu as pltpu
from jax.experimental.pallas import tpu_sc as plsc

NUM_FIELDS = 26
EMBED_DIM = 32
BATCH = 4096
VOCAB = 100000
OUT_DIM = NUM_FIELDS * EMBED_DIM
NR = 144
RPS = 9
FPC = NUM_FIELDS // 2
GROUPS = (7, 6)
GMAX = 7
FROW = BATCH * EMBED_DIM
STAGE_W = 896

VB = [(r * (VOCAB // NR)) // 128 * 128 for r in range(NR)] + [VOCAB]


@functools.cache
def _build():
    mesh = plsc.VectorSubcoreMesh(core_axis_name="c", subcore_axis_name="s")

    @functools.partial(
        pl.kernel,
        mesh=mesh,
        out_type=jax.ShapeDtypeStruct((OUT_DIM * BATCH,), jnp.float32),
        scratch_types=[
            pltpu.VMEM((4, 8, STAGE_W), jnp.float32),
            pltpu.VMEM((128,), jnp.int32),
            pltpu.VMEM((1, 216), jnp.int32),
            pltpu.VMEM((EMBED_DIM * 128,), jnp.float32),
            pltpu.VMEM((EMBED_DIM * 128,), jnp.int32),
            pltpu.VMEM_SHARED((GMAX * FROW,), jnp.float32),
            pltpu.SemaphoreType.DMA,
            pltpu.SemaphoreType.DMA,
        ],
        compiler_params=pltpu.CompilerParams(needs_layout_passes=False),
    )
    def k(skey_hbm, b3_hbm, *args):
        tables = args[:NUM_FIELDS]
        (out_hbm, stg, skb, b3v, data_v, idx_v, grp_sh,
         sem_stage, sem_sc) = args[NUM_FIELDS:]
        cid = lax.axis_index("c")
        sid = lax.axis_index("s")
        lane = lax.iota(jnp.int32, 16)

        def init_idx(t, _):
            idx_v[pl.ds(t * 16, 16)] = jnp.full((16,), -1, jnp.int32)
            return 0

        lax.fori_loop(0, EMBED_DIM * 8, init_idx, 0)

        def scatter_args():
            return (
                data_v,
                grp_sh.at[plsc.Indices(idx_v, ignored_value=-1)],
                sem_sc,
            )

        def do_phase(i, fi, r):
            lo = (r * (VOCAB // NR)) // 128 * 128
            start = pl.multiple_of(lo, 128)
            st_copies = [
                pltpu.async_copy(
                    tables[i].at[pl.ds(dg * 8, 8), pl.ds(start, STAGE_W)],
                    stg.at[dg],
                    sem_stage,
                )
                for dg in range(4)
            ]
            for c_ in st_copies:
                c_.wait()

            pltpu.sync_copy(b3_hbm.at[r], b3v)
            vec = b3v[0, pl.ds(i * 8, 16)]
            j0 = vec[0]
            j1 = vec[1]
            jbase = j0 & jnp.int32(-128)
            nch = (j1 - jbase + 127) >> 7
            obase_i = jnp.int32(fi * FROW)

            def chunk_body(c, _):
                jb = pl.multiple_of(jbase + c * 128, 128)
                pltpu.sync_copy(skey_hbm.at[i, 0, pl.ds(jb, 128)], skb)
                q0 = jnp.maximum((j0 - jb) >> 4, 0)
                q1 = jnp.minimum((j1 - jb + 15) >> 4, 8)

                def q_body(q, _):
                    kv = skb[pl.ds(q * 16, 16)]
                    vv = lax.shift_right_logical(kv, 12)
                    bb = kv & jnp.int32(4095)
                    jl = jb + q * 16 + lane
                    mask = (jl >= j0) & (jl < j1)
                    cols = jnp.clip(vv - start, 0, STAGE_W - 1)
                    ob = obase_i + bb

                    def d_body(d, _):
                        dg = jnp.full((16,), d >> 3, jnp.int32)
                        dr = jnp.full((16,), d & 7, jnp.int32)
                        g = plsc.load_gather(stg, [dg, dr, cols])
                        sid_ = ob + d * BATCH
                        sid_ = jnp.where(mask, sid_, jnp.int32(-1))
                        data_v[pl.ds(d * 128 + q * 16, 16)] = g
                        idx_v[pl.ds(d * 128 + q * 16, 16)] = sid_
                        return 0

                    lax.fori_loop(0, EMBED_DIM, d_body, 0)
                    return 0

                lax.fori_loop(q0, q1, q_body, 0)
                pltpu.async_copy(*scatter_args()).wait()
                return 0

            lax.fori_loop(0, nch, chunk_body, 0)

        fbase = 0
        for gsz in GROUPS:
            for c_id in range(2):

                @pl.when(cid == c_id)
                def _(c_id=c_id, fbase=fbase, gsz=gsz):
                    for fi in range(gsz):
                        i = c_id * FPC + fbase + fi

                        def rr_body(hr, _, i=i, fi=fi):
                            for pp in range(2):
                                rr = 2 * hr + pp

                                @pl.when(rr < RPS)
                                def _(rr=rr):
                                    do_phase(i, fi, sid * RPS + rr)

                            return 0

                        lax.fori_loop(0, (RPS + 1) >> 1, rr_body, 0)

            plsc.subcore_barrier()
            gelems = gsz * FROW
            sl = gelems // 16
            src_off = sid * sl
            dst_off = (cid * FPC + fbase) * FROW + sid * sl
            pltpu.sync_copy(
                grp_sh.at[pl.ds(src_off, sl)],
                out_hbm.at[pl.ds(dst_off, sl)],
            )
            plsc.subcore_barrier()
            fbase += gsz

    return k


def kernel(*args):
    feats = args[:NUM_FIELDS]
    tables = args[NUM_FIELDS:]
    idx = jnp.stack(feats)
    b_arr = jnp.arange(BATCH, dtype=jnp.int32)
    skey = jnp.sort(idx * BATCH + b_arr[None, :], axis=1)
    skey_p = jnp.pad(
        skey, ((0, 0), (0, 128)), constant_values=2**31 - 1
    )[:, None, :]
    vbk = jnp.array([v * BATCH for v in VB], dtype=jnp.int32)
    bounds = jax.vmap(lambda row: jnp.searchsorted(row, vbk))(skey)
    bounds = bounds.astype(jnp.int32)
    b3 = jnp.zeros((NR, 1, 216), jnp.int32)
    b3 = b3.at[:, 0, 0 : 8 * NUM_FIELDS : 8].set(bounds[:, :NR].T)
    b3 = b3.at[:, 0, 1 : 8 * NUM_FIELDS : 8].set(bounds[:, 1 : NR + 1].T)
    out1 = _build()(skey_p, b3, *[t.T for t in tables])
    return out1.reshape(OUT_DIM, BATCH).T


# speedup vs baseline: 1.5489x; 1.5489x over previous
import functools

import jax
import jax.numpy as jnp
from jax import lax
from jax.experimental import pallas as pl
from jax.experimental.pallas import tpu as pltpu
from jax.experimental.pallas import tpu_sc as plsc

NUM_FIELDS = 26
EMBED_DIM = 32
BATCH = 4096
VOCAB = 100000
OUT_DIM = NUM_FIELDS * EMBED_DIM
NR = 80
RPS = 5
FPC = NUM_FIELDS // 2
GROUPS = (7, 6)
GMAX = 7
FROW = BATCH * EMBED_DIM
STAGE_W = 1408

VB = [(r * (VOCAB // NR)) // 128 * 128 for r in range(NR)] + [VOCAB]


@functools.cache
def _build():
    mesh = plsc.VectorSubcoreMesh(core_axis_name="c", subcore_axis_name="s")

    @functools.partial(
        pl.kernel,
        mesh=mesh,
        out_type=jax.ShapeDtypeStruct((OUT_DIM * BATCH,), jnp.float32),
        scratch_types=[
            pltpu.VMEM((4, 8, STAGE_W), jnp.float32),
            pltpu.VMEM((128,), jnp.int32),
            pltpu.VMEM((RPS, 1, 216), jnp.int32),
            pltpu.VMEM((EMBED_DIM * 128,), jnp.float32),
            pltpu.VMEM((EMBED_DIM * 128,), jnp.int32),
            pltpu.VMEM_SHARED((GMAX * FROW,), jnp.float32),
            pltpu.SemaphoreType.DMA,
            pltpu.SemaphoreType.DMA,
        ],
        compiler_params=pltpu.CompilerParams(needs_layout_passes=False),
    )
    def k(skey_hbm, b3_hbm, *args):
        tables = args[:NUM_FIELDS]
        (out_hbm, stg, skb, b3v, data_v, idx_v, grp_sh,
         sem_stage, sem_sc) = args[NUM_FIELDS:]
        cid = lax.axis_index("c")
        sid = lax.axis_index("s")
        lane = lax.iota(jnp.int32, 16)

        def init_idx(t, _):
            idx_v[pl.ds(t * 16, 16)] = jnp.full((16,), -1, jnp.int32)
            return 0

        lax.fori_loop(0, EMBED_DIM * 8, init_idx, 0)
        pltpu.sync_copy(b3_hbm.at[pl.ds(sid * RPS, RPS)], b3v)

        def scatter_args():
            return (
                data_v,
                grp_sh.at[plsc.Indices(idx_v, ignored_value=-1)],
                sem_sc,
            )

        def do_phase(i, fi, rr):
            r = sid * RPS + rr
            lo = (r * (VOCAB // NR)) // 128 * 128
            start = pl.multiple_of(lo, 128)
            st_copies = [
                pltpu.async_copy(
                    tables[i].at[pl.ds(dg * 8, 8), pl.ds(start, STAGE_W)],
                    stg.at[dg],
                    sem_stage,
                )
                for dg in range(4)
            ]
            for c_ in st_copies:
                c_.wait()

            vec = b3v[rr, 0, pl.ds(i * 8, 16)]
            j0 = vec[0]
            j1 = vec[1]
            jbase = j0 & jnp.int32(-128)
            nch = (j1 - jbase + 127) >> 7
            obase_i = jnp.int32(fi * FROW)

            def chunk_body(c, _):
                jb = pl.multiple_of(jbase + c * 128, 128)
                pltpu.sync_copy(skey_hbm.at[i, 0, pl.ds(jb, 128)], skb)
                q0 = jnp.maximum((j0 - jb) >> 4, 0)
                q1 = jnp.minimum((j1 - jb + 15) >> 4, 8)

                def q_body(q, _):
                    kv = skb[pl.ds(q * 16, 16)]
                    vv = lax.shift_right_logical(kv, 12)
                    bb = kv & jnp.int32(4095)
                    jl = jb + q * 16 + lane
                    mask = (jl >= j0) & (jl < j1)
                    cols = jnp.clip(vv - start, 0, STAGE_W - 1)
                    ob = obase_i + bb

                    def d_body(d, _):
                        dg = jnp.full((16,), d >> 3, jnp.int32)
                        dr = jnp.full((16,), d & 7, jnp.int32)
                        g = plsc.load_gather(stg, [dg, dr, cols])
                        sid_ = ob + d * BATCH
                        sid_ = jnp.where(mask, sid_, jnp.int32(-1))
                        data_v[pl.ds(d * 128 + q * 16, 16)] = g
                        idx_v[pl.ds(d * 128 + q * 16, 16)] = sid_
                        return 0

                    lax.fori_loop(0, EMBED_DIM, d_body, 0)
                    return 0

                lax.fori_loop(q0, q1, q_body, 0)
                pltpu.async_copy(*scatter_args()).wait()
                return 0

            lax.fori_loop(0, nch, chunk_body, 0)

        fbase = 0
        for gsz in GROUPS:
            for c_id in range(2):

                @pl.when(cid == c_id)
                def _(c_id=c_id, fbase=fbase, gsz=gsz):
                    for fi in range(gsz):
                        i = c_id * FPC + fbase + fi

                        def rr_body(hr, _, i=i, fi=fi):
                            for pp in range(2):
                                rr = 2 * hr + pp

                                @pl.when(rr < RPS)
                                def _(rr=rr):
                                    do_phase(i, fi, rr)

                            return 0

                        lax.fori_loop(0, (RPS + 1) >> 1, rr_body, 0)

            plsc.subcore_barrier()
            gelems = gsz * FROW
            sl = gelems // 16
            src_off = sid * sl
            dst_off = (cid * FPC + fbase) * FROW + sid * sl
            pltpu.sync_copy(
                grp_sh.at[pl.ds(src_off, sl)],
                out_hbm.at[pl.ds(dst_off, sl)],
            )
            plsc.subcore_barrier()
            fbase += gsz

    return k


def kernel(*args):
    feats = args[:NUM_FIELDS]
    tables = args[NUM_FIELDS:]
    idx = jnp.stack(feats)
    b_arr = jnp.arange(BATCH, dtype=jnp.int32)
    skey = jnp.sort(idx * BATCH + b_arr[None, :], axis=1)
    skey_p = jnp.pad(
        skey, ((0, 0), (0, 128)), constant_values=2**31 - 1
    )[:, None, :]
    vbk = jnp.array([v * BATCH for v in VB], dtype=jnp.int32)
    bounds = jax.vmap(lambda row: jnp.searchsorted(row, vbk))(skey)
    bounds = bounds.astype(jnp.int32)
    b3 = jnp.zeros((NR, 1, 216), jnp.int32)
    b3 = b3.at[:, 0, 0 : 8 * NUM_FIELDS : 8].set(bounds[:, :NR].T)
    b3 = b3.at[:, 0, 1 : 8 * NUM_FIELDS : 8].set(bounds[:, 1 : NR + 1].T)
    out1 = _build()(skey_p, b3, *[t.T for t in tables])
    return out1.reshape(OUT_DIM, BATCH).T
